# Initial kernel scaffold; baseline (speedup 1.0000x reference)
import jax
import jax.numpy as jnp
from jax.experimental import pallas as pl


def kernel(x, edge_index, W1, b1, W2, b2):
    raise NotImplementedError("write your pallas kernel here")


# speedup vs baseline: 29.2079x; 29.2079x over previous
import functools

import jax
import jax.numpy as jnp
from jax import lax
from jax.experimental import pallas as pl
from jax.experimental.pallas import tpu as pltpu
from jax.experimental.pallas import tpu_sc as plsc

N = 10000
H = 16
NC, NS = 2, 16
NW = NC * NS
NP = 10240
RPS = NP // NS
E = 320000
EP = 327680
CHUNK = 128
NCH = EP // NW // CHUNK
EROWS = EP // CHUNK

_mesh = plsc.VectorSubcoreMesh(core_axis_name="c", subcore_axis_name="s")
_sc_params = pltpu.CompilerParams(use_tc_tiling_on_sc=False)


@functools.partial(
    pl.kernel,
    out_type=jax.ShapeDtypeStruct((NC * NP, H), jnp.float32),
    mesh=_mesh,
    compiler_params=_sc_params,
    scratch_types=[
        pltpu.VMEM((NCH, CHUNK), jnp.int32),
        pltpu.VMEM((CHUNK, H), jnp.float32),
        pltpu.VMEM_SHARED((NP, H), jnp.float32),
    ],
)
def _sc_degree(dst_hbm, ones_hbm, zeros_hbm, out_hbm, dst_v, ones_v, acc_sh):
    c = lax.axis_index("c")
    s = lax.axis_index("s")
    wid = c * NS + s
    pltpu.sync_copy(zeros_hbm.at[pl.ds(s * RPS, RPS)],
                    acc_sh.at[pl.ds(s * RPS, RPS)])
    pltpu.sync_copy(dst_hbm.at[pl.ds(wid * NCH, NCH)], dst_v)
    pltpu.sync_copy(ones_hbm, ones_v)
    plsc.subcore_barrier()

    def body(i, carry):
        pltpu.sync_copy(ones_v, acc_sh.at[dst_v.at[i]], add=True)
        return carry

    lax.fori_loop(0, NCH, body, 0)
    plsc.subcore_barrier()
    pltpu.sync_copy(acc_sh.at[pl.ds(s * RPS, RPS)],
                    out_hbm.at[pl.ds(c * NP + s * RPS, RPS)])


@functools.partial(
    pl.kernel,
    out_type=jax.ShapeDtypeStruct((NC * NP, H), jnp.float32),
    mesh=_mesh,
    compiler_params=_sc_params,
    scratch_types=[
        pltpu.VMEM((NCH, CHUNK), jnp.int32),
        pltpu.VMEM((NCH, CHUNK), jnp.int32),
        pltpu.VMEM((CHUNK, H), jnp.float32),
        pltpu.VMEM_SHARED((NP, H), jnp.float32),
    ],
)
def _sc_propagate(hp_hbm, src_hbm, dst_hbm, zeros_hbm, out_hbm,
                  src_v, dst_v, rows_v, acc_sh):
    c = lax.axis_index("c")
    s = lax.axis_index("s")
    wid = c * NS + s
    pltpu.sync_copy(zeros_hbm.at[pl.ds(s * RPS, RPS)],
                    acc_sh.at[pl.ds(s * RPS, RPS)])
    pltpu.sync_copy(src_hbm.at[pl.ds(wid * NCH, NCH)], src_v)
    pltpu.sync_copy(dst_hbm.at[pl.ds(wid * NCH, NCH)], dst_v)
    plsc.subcore_barrier()

    def body(i, carry):
        pltpu.sync_copy(hp_hbm.at[src_v.at[i]], rows_v)
        pltpu.sync_copy(rows_v, acc_sh.at[dst_v.at[i]], add=True)
        return carry

    lax.fori_loop(0, NCH, body, 0)
    plsc.subcore_barrier()
    pltpu.sync_copy(acc_sh.at[pl.ds(s * RPS, RPS)],
                    out_hbm.at[pl.ds(c * NP + s * RPS, RPS)])


R = 1000
G = N // R


def _tc_first_body(x_ref, w1_ref, d0_ref, d1_ref, hp_ref):
    dis = lax.rsqrt(d0_ref[...] + d1_ref[...] + 1.0)
    h = jnp.dot(x_ref[...], w1_ref[...], preferred_element_type=jnp.float32)
    hp_ref[...] = dis * h


def _tc_mid_body(e0_ref, e1_ref, hp1_ref, d0_ref, d1_ref, b1_ref, out_ref):
    dis = lax.rsqrt(d0_ref[...] + d1_ref[...] + 1.0)
    acc = e0_ref[...] + e1_ref[...] + hp1_ref[...]
    h1 = jnp.maximum(dis * acc + b1_ref[...], 0.0)
    out_ref[...] = dis * h1


def _tc_final_body(e0_ref, e1_ref, hp2_ref, d0_ref, d1_ref, w2_ref, b2_ref,
                   out_ref):
    dis = lax.rsqrt(d0_ref[...] + d1_ref[...] + 1.0)
    t = dis * (e0_ref[...] + e1_ref[...] + hp2_ref[...])
    out_ref[...] = (jnp.dot(t, w2_ref[...], preferred_element_type=jnp.float32)
                    + b2_ref[...])


def _row_spec(w):
    return pl.BlockSpec((R, w), lambda i: (i, 0))


def _full_spec(h, w):
    return pl.BlockSpec((h, w), lambda i: (0, 0))


def _tc_first(x, W1, d0, d1):
    return pl.pallas_call(
        _tc_first_body,
        grid=(G,),
        in_specs=[_row_spec(128), _full_spec(128, H), _row_spec(H),
                  _row_spec(H)],
        out_specs=_row_spec(H),
        out_shape=jax.ShapeDtypeStruct((N, H), jnp.float32),
    )(x, W1, d0, d1)


def _tc_mid(e0, e1, hp1, d0, d1, b1):
    return pl.pallas_call(
        _tc_mid_body,
        grid=(G,),
        in_specs=[_row_spec(H)] * 5 + [_full_spec(1, H)],
        out_specs=_row_spec(H),
        out_shape=jax.ShapeDtypeStruct((N, H), jnp.float32),
    )(e0, e1, hp1, d0, d1, b1)


def _tc_final(e0, e1, hp2, d0, d1, W2, b2):
    return pl.pallas_call(
        _tc_final_body,
        grid=(G,),
        in_specs=[_row_spec(H)] * 5 + [_full_spec(H, 128), _full_spec(1, 128)],
        out_specs=_row_spec(128),
        out_shape=jax.ShapeDtypeStruct((N, 128), jnp.float32),
    )(e0, e1, hp2, d0, d1, W2, b2)


def kernel(x, edge_index, W1, b1, W2, b2):
    ei = edge_index.astype(jnp.int32)
    pad = jnp.full((EP - E,), NP - 1, jnp.int32)
    src = jnp.concatenate([ei[0], pad]).reshape(EROWS, CHUNK)
    dst = jnp.concatenate([ei[1], pad]).reshape(EROWS, CHUNK)
    zeros_p = jnp.zeros((NP, H), jnp.float32)
    ones_c = jnp.ones((CHUNK, H), jnp.float32)

    degp = _sc_degree(dst, ones_c, zeros_p)
    d0 = degp[:N]
    d1 = degp[NP:NP + N]

    hp1 = _tc_first(x, W1, d0, d1)
    e1 = _sc_propagate(jnp.pad(hp1, ((0, NP - N), (0, 0))), src, dst, zeros_p)
    hp2 = _tc_mid(e1[:N], e1[NP:NP + N], hp1, d0, d1, b1.reshape(1, H))
    e2 = _sc_propagate(jnp.pad(hp2, ((0, NP - N), (0, 0))), src, dst, zeros_p)
    return _tc_final(e2[:N], e2[NP:NP + N], hp2, d0, d1, W2,
                     b2.reshape(1, 128))


# speedup vs baseline: 55.6622x; 1.9057x over previous
import functools

import jax
import jax.numpy as jnp
from jax import lax
from jax.experimental import pallas as pl
from jax.experimental.pallas import tpu as pltpu
from jax.experimental.pallas import tpu_sc as plsc

N = 10000
H = 16
NC, NS = 2, 16
NW = NC * NS
NP = 10240
RPS = NP // NS
E = 320000
CH = 128
NCHT = E // CH
CPW = NCHT // NW
GRP = 6
NG = CPW // GRP
GR = GRP * CH

_mesh = plsc.VectorSubcoreMesh(core_axis_name="c", subcore_axis_name="s")
_sc_params = pltpu.CompilerParams(use_tc_tiling_on_sc=False)

_partial_out = (jax.ShapeDtypeStruct((NP, H), jnp.float32),
                jax.ShapeDtypeStruct((NP, H), jnp.float32))


@functools.partial(
    pl.kernel,
    out_type=_partial_out,
    mesh=_mesh,
    compiler_params=_sc_params,
    scratch_types=[
        pltpu.VMEM((CPW, CH), jnp.int32),
        pltpu.VMEM((1, CH), jnp.int32),
        pltpu.VMEM((CH, H), jnp.float32),
        pltpu.VMEM_SHARED((NP, H), jnp.float32),
    ],
)
def _sc_degree(dst_hbm, ones_hbm, zeros_hbm, out0_hbm, out1_hbm,
               dst_v, dste_v, ones_v, acc_sh):
    c = lax.axis_index("c")
    s = lax.axis_index("s")
    w = c * NS + s
    pltpu.sync_copy(zeros_hbm.at[pl.ds(s * RPS, RPS)],
                    acc_sh.at[pl.ds(s * RPS, RPS)])
    pltpu.sync_copy(dst_hbm.at[pl.ds(w * CPW, CPW)], dst_v)
    pltpu.sync_copy(ones_hbm, ones_v)

    @pl.when(w < NCHT - NW * CPW)
    def _():
        pltpu.sync_copy(dst_hbm.at[pl.ds(NW * CPW + w, 1)], dste_v)

    plsc.subcore_barrier()

    def body(i, carry):
        pltpu.sync_copy(ones_v, acc_sh.at[dst_v.at[i]], add=True)
        return carry

    lax.fori_loop(0, CPW, body, 0)

    @pl.when(w < NCHT - NW * CPW)
    def _():
        pltpu.sync_copy(ones_v, acc_sh.at[dste_v.at[0]], add=True)

    plsc.subcore_barrier()

    @pl.when(c == 0)
    def _():
        pltpu.sync_copy(acc_sh.at[pl.ds(s * RPS, RPS)],
                        out0_hbm.at[pl.ds(s * RPS, RPS)])

    @pl.when(c == 1)
    def _():
        pltpu.sync_copy(acc_sh.at[pl.ds(s * RPS, RPS)],
                        out1_hbm.at[pl.ds(s * RPS, RPS)])


@functools.partial(
    pl.kernel,
    out_type=_partial_out,
    mesh=_mesh,
    compiler_params=_sc_params,
    scratch_types=[
        pltpu.VMEM((CPW, CH), jnp.int32),
        pltpu.VMEM((CPW, CH), jnp.int32),
        pltpu.VMEM((1, CH), jnp.int32),
        pltpu.VMEM((1, CH), jnp.int32),
        pltpu.VMEM((2, GR, H), jnp.float32),
        pltpu.VMEM_SHARED((NP, H), jnp.float32),
        pltpu.SemaphoreType.DMA((2,)),
    ],
)
def _sc_propagate(hp_hbm, src_hbm, dst_hbm, zeros_hbm, out0_hbm, out1_hbm,
                  src_v, dst_v, srce_v, dste_v, rows_v, acc_sh, gsem):
    c = lax.axis_index("c")
    s = lax.axis_index("s")
    w = c * NS + s
    pltpu.sync_copy(zeros_hbm.at[pl.ds(s * RPS, RPS)],
                    acc_sh.at[pl.ds(s * RPS, RPS)])
    pltpu.sync_copy(src_hbm.at[pl.ds(w * CPW, CPW)], src_v)
    pltpu.sync_copy(dst_hbm.at[pl.ds(w * CPW, CPW)], dst_v)

    @pl.when(w < NCHT - NW * CPW)
    def _():
        pltpu.sync_copy(src_hbm.at[pl.ds(NW * CPW + w, 1)], srce_v)
        pltpu.sync_copy(dst_hbm.at[pl.ds(NW * CPW + w, 1)], dste_v)

    plsc.subcore_barrier()

    def _gather(g, b):
        for k in range(GRP):
            pltpu.async_copy(hp_hbm.at[src_v.at[g * GRP + k]],
                             rows_v.at[b].at[pl.ds(k * CH, CH)],
                             gsem.at[b])

    def _wait_gather(g, b):
        for k in range(GRP):
            pltpu.make_async_copy(hp_hbm.at[src_v.at[g * GRP + k]],
                                  rows_v.at[b].at[pl.ds(k * CH, CH)],
                                  gsem.at[b]).wait()

    _gather(0, 0)

    def body(g, carry):
        b = lax.rem(g, 2)
        _wait_gather(g, b)

        @pl.when(g + 1 < NG)
        def _():
            _gather(g + 1, 1 - b)

        for k in range(GRP):
            pltpu.sync_copy(rows_v.at[b].at[pl.ds(k * CH, CH)],
                            acc_sh.at[dst_v.at[g * GRP + k]], add=True)
        return carry

    lax.fori_loop(0, NG, body, 0)

    @pl.when(w < NCHT - NW * CPW)
    def _():
        pltpu.async_copy(hp_hbm.at[srce_v.at[0]],
                         rows_v.at[0].at[pl.ds(0, CH)], gsem.at[0])
        pltpu.make_async_copy(hp_hbm.at[srce_v.at[0]],
                              rows_v.at[0].at[pl.ds(0, CH)],
                              gsem.at[0]).wait()
        pltpu.sync_copy(rows_v.at[0].at[pl.ds(0, CH)],
                        acc_sh.at[dste_v.at[0]], add=True)

    plsc.subcore_barrier()

    @pl.when(c == 0)
    def _():
        pltpu.sync_copy(acc_sh.at[pl.ds(s * RPS, RPS)],
                        out0_hbm.at[pl.ds(s * RPS, RPS)])

    @pl.when(c == 1)
    def _():
        pltpu.sync_copy(acc_sh.at[pl.ds(s * RPS, RPS)],
                        out1_hbm.at[pl.ds(s * RPS, RPS)])


R = 2000
G = N // R


def _tc_first_body(x_ref, w1_ref, d0_ref, d1_ref, hp_ref):
    dis = lax.rsqrt(d0_ref[...] + d1_ref[...] + 1.0)
    h = jnp.dot(x_ref[...], w1_ref[...], preferred_element_type=jnp.float32)
    hp_ref[...] = dis * h


def _tc_mid_body(e0_ref, e1_ref, hp1_ref, d0_ref, d1_ref, b1_ref, out_ref):
    dis = lax.rsqrt(d0_ref[...] + d1_ref[...] + 1.0)
    acc = e0_ref[...] + e1_ref[...] + hp1_ref[...]
    h1 = jnp.maximum(dis * acc + b1_ref[...], 0.0)
    out_ref[...] = dis * h1


def _tc_final_body(e0_ref, e1_ref, hp2_ref, d0_ref, d1_ref, w2_ref, b2_ref,
                   out_ref):
    dis = lax.rsqrt(d0_ref[...] + d1_ref[...] + 1.0)
    t = dis * (e0_ref[...] + e1_ref[...] + hp2_ref[...])
    out_ref[...] = (jnp.dot(t, w2_ref[...], preferred_element_type=jnp.float32)
                    + b2_ref[...])


def _row_spec(width):
    return pl.BlockSpec((R, width), lambda i: (i, 0))


def _full_spec(h, width):
    return pl.BlockSpec((h, width), lambda i: (0, 0))


def _tc_first(x, W1, d0, d1):
    return pl.pallas_call(
        _tc_first_body,
        grid=(G,),
        in_specs=[_row_spec(128), _full_spec(128, H), _row_spec(H),
                  _row_spec(H)],
        out_specs=_row_spec(H),
        out_shape=jax.ShapeDtypeStruct((NP, H), jnp.float32),
    )(x, W1, d0, d1)


def _tc_mid(e0, e1, hp1, d0, d1, b1):
    return pl.pallas_call(
        _tc_mid_body,
        grid=(G,),
        in_specs=[_row_spec(H)] * 5 + [_full_spec(1, H)],
        out_specs=_row_spec(H),
        out_shape=jax.ShapeDtypeStruct((NP, H), jnp.float32),
    )(e0, e1, hp1, d0, d1, b1)


def _tc_final(e0, e1, hp2, d0, d1, W2, b2):
    return pl.pallas_call(
        _tc_final_body,
        grid=(G,),
        in_specs=[_row_spec(H)] * 5 + [_full_spec(H, 128), _full_spec(1, 128)],
        out_specs=_row_spec(128),
        out_shape=jax.ShapeDtypeStruct((N, 128), jnp.float32),
    )(e0, e1, hp2, d0, d1, W2, b2)


def kernel(x, edge_index, W1, b1, W2, b2):
    ei = edge_index.astype(jnp.int32)
    src = ei[0].reshape(NCHT, CH)
    dst = ei[1].reshape(NCHT, CH)
    zeros_p = jnp.zeros((NP, H), jnp.float32)
    ones_c = jnp.ones((CH, H), jnp.float32)

    d0, d1 = _sc_degree(dst, ones_c, zeros_p)
    hp1 = _tc_first(x, W1, d0, d1)
    e10, e11 = _sc_propagate(hp1, src, dst, zeros_p)
    hp2 = _tc_mid(e10, e11, hp1, d0, d1, b1.reshape(1, H))
    e20, e21 = _sc_propagate(hp2, src, dst, zeros_p)
    return _tc_final(e20, e21, hp2, d0, d1, W2, b2.reshape(1, 128))


# speedup vs baseline: 65.0116x; 1.1680x over previous
import functools

import jax
import jax.numpy as jnp
from jax import lax
from jax.experimental import pallas as pl
from jax.experimental.pallas import tpu as pltpu
from jax.experimental.pallas import tpu_sc as plsc

N = 10000
H = 16
NC, NS = 2, 16
NW = NC * NS
NP = 10240
RPS = NP // NS
E = 320000
CH = 128
NCHT = E // CH
CPW = NCHT // NW
NX = NCHT - NW * CPW
GRP = 6
NG = CPW // GRP
GR = GRP * CH

_mesh = plsc.VectorSubcoreMesh(core_axis_name="c", subcore_axis_name="s")
_sc_params = pltpu.CompilerParams(use_tc_tiling_on_sc=False)

_partial_out = (jax.ShapeDtypeStruct((NP, H), jnp.float32),
                jax.ShapeDtypeStruct((NP, H), jnp.float32))


@functools.partial(
    pl.kernel,
    out_type=_partial_out,
    mesh=_mesh,
    compiler_params=_sc_params,
    scratch_types=[
        pltpu.VMEM((CPW, CH), jnp.int32),
        pltpu.VMEM((1, CH), jnp.int32),
        pltpu.VMEM((CH, H), jnp.float32),
        pltpu.VMEM_SHARED((NP, H), jnp.float32),
        pltpu.SemaphoreType.DMA((2,)),
    ],
)
def _sc_degree(dst_hbm, ones_hbm, zeros_hbm, out0_hbm, out1_hbm,
               dst_v, dste_v, ones_v, acc_sh, ssem):
    c = lax.axis_index("c")
    s = lax.axis_index("s")
    w = c * NS + s
    pltpu.sync_copy(zeros_hbm.at[pl.ds(s * RPS, RPS)],
                    acc_sh.at[pl.ds(s * RPS, RPS)])
    pltpu.sync_copy(dst_hbm.at[pl.ds(w * CPW, CPW)], dst_v)
    pltpu.sync_copy(ones_hbm, ones_v)

    @pl.when(w < NX)
    def _():
        pltpu.sync_copy(dst_hbm.at[pl.ds(NW * CPW + w, 1)], dste_v)

    plsc.subcore_barrier()

    def body(j, carry):
        b = lax.rem(j, 2)
        for k in range(GRP):
            pltpu.async_copy(ones_v, acc_sh.at[dst_v.at[j * GRP + k]],
                             ssem.at[b], add=True)

        @pl.when(j >= 1)
        def _():
            for k in range(GRP):
                pltpu.make_async_copy(
                    ones_v, acc_sh.at[dst_v.at[(j - 1) * GRP + k]],
                    ssem.at[1 - b]).wait()

        return carry

    lax.fori_loop(0, NG, body, 0)
    for k in range(GRP):
        pltpu.make_async_copy(ones_v,
                              acc_sh.at[dst_v.at[(NG - 1) * GRP + k]],
                              ssem.at[(NG - 1) % 2]).wait()

    @pl.when(w < NX)
    def _():
        pltpu.sync_copy(ones_v, acc_sh.at[dste_v.at[0]], add=True)

    plsc.subcore_barrier()

    @pl.when(c == 0)
    def _():
        pltpu.sync_copy(acc_sh.at[pl.ds(s * RPS, RPS)],
                        out0_hbm.at[pl.ds(s * RPS, RPS)])

    @pl.when(c == 1)
    def _():
        pltpu.sync_copy(acc_sh.at[pl.ds(s * RPS, RPS)],
                        out1_hbm.at[pl.ds(s * RPS, RPS)])


def _edge_phase(gather_src, src_v, dst_v, srce_v, dste_v, rows_v, acc_sh,
                gsem, ssem, w):
    def _issue_gathers(g, b):
        for k in range(GRP):
            pltpu.async_copy(gather_src.at[src_v.at[g * GRP + k]],
                             rows_v.at[b].at[pl.ds(k * CH, CH)],
                             gsem.at[b])

    def _wait_gathers(g, b):
        for k in range(GRP):
            pltpu.make_async_copy(gather_src.at[src_v.at[g * GRP + k]],
                                  rows_v.at[b].at[pl.ds(k * CH, CH)],
                                  gsem.at[b]).wait()

    def _issue_scatters(g, b):
        for k in range(GRP):
            pltpu.async_copy(rows_v.at[b].at[pl.ds(k * CH, CH)],
                             acc_sh.at[dst_v.at[g * GRP + k]],
                             ssem.at[b], add=True)

    def _drain_scatters(g, b):
        for k in range(GRP):
            pltpu.make_async_copy(rows_v.at[b].at[pl.ds(k * CH, CH)],
                                  acc_sh.at[dst_v.at[g * GRP + k]],
                                  ssem.at[b]).wait()

    _issue_gathers(0, 0)

    def body(g, carry):
        b = lax.rem(g, 2)
        _wait_gathers(g, b)

        @pl.when(g >= 1)
        def _():
            _drain_scatters(g - 1, 1 - b)

        @pl.when(g + 1 < NG)
        def _():
            _issue_gathers(g + 1, 1 - b)

        _issue_scatters(g, b)
        return carry

    lax.fori_loop(0, NG, body, 0)
    _drain_scatters(NG - 1, (NG - 1) % 2)

    @pl.when(w < NX)
    def _():
        pltpu.sync_copy(gather_src.at[srce_v.at[0]],
                        rows_v.at[0].at[pl.ds(0, CH)])
        pltpu.sync_copy(rows_v.at[0].at[pl.ds(0, CH)],
                        acc_sh.at[dste_v.at[0]], add=True)


_prop_scratch = [
    pltpu.VMEM((CPW, CH), jnp.int32),
    pltpu.VMEM((CPW, CH), jnp.int32),
    pltpu.VMEM((1, CH), jnp.int32),
    pltpu.VMEM((1, CH), jnp.int32),
    pltpu.VMEM((2, GR, H), jnp.float32),
    pltpu.VMEM_SHARED((NP, H), jnp.float32),
    pltpu.SemaphoreType.DMA((2,)),
    pltpu.SemaphoreType.DMA((2,)),
]


@functools.partial(
    pl.kernel,
    out_type=_partial_out,
    mesh=_mesh,
    compiler_params=_sc_params,
    scratch_types=_prop_scratch,
)
def _sc_prop1(hp_hbm, src_hbm, dst_hbm, zeros_hbm, out0_hbm, out1_hbm,
              src_v, dst_v, srce_v, dste_v, rows_v, acc_sh, gsem, ssem):
    c = lax.axis_index("c")
    s = lax.axis_index("s")
    w = c * NS + s
    @pl.when(c == 0)
    def _():
        pltpu.sync_copy(hp_hbm.at[pl.ds(s * RPS, RPS)],
                        acc_sh.at[pl.ds(s * RPS, RPS)])

    @pl.when(c == 1)
    def _():
        pltpu.sync_copy(zeros_hbm.at[pl.ds(s * RPS, RPS)],
                        acc_sh.at[pl.ds(s * RPS, RPS)])

    pltpu.sync_copy(src_hbm.at[pl.ds(w * CPW, CPW)], src_v)
    pltpu.sync_copy(dst_hbm.at[pl.ds(w * CPW, CPW)], dst_v)

    @pl.when(w < NX)
    def _():
        pltpu.sync_copy(src_hbm.at[pl.ds(NW * CPW + w, 1)], srce_v)
        pltpu.sync_copy(dst_hbm.at[pl.ds(NW * CPW + w, 1)], dste_v)

    plsc.subcore_barrier()
    _edge_phase(hp_hbm, src_v, dst_v, srce_v, dste_v, rows_v, acc_sh,
                gsem, ssem, w)
    plsc.subcore_barrier()

    @pl.when(c == 0)
    def _():
        pltpu.sync_copy(acc_sh.at[pl.ds(s * RPS, RPS)],
                        out0_hbm.at[pl.ds(s * RPS, RPS)])

    @pl.when(c == 1)
    def _():
        pltpu.sync_copy(acc_sh.at[pl.ds(s * RPS, RPS)],
                        out1_hbm.at[pl.ds(s * RPS, RPS)])


@functools.partial(
    pl.kernel,
    out_type=_partial_out,
    mesh=_mesh,
    compiler_params=_sc_params,
    scratch_types=_prop_scratch + [
        pltpu.VMEM((RPS, H), jnp.float32),
        pltpu.VMEM((RPS, H), jnp.float32),
        pltpu.VMEM((RPS, H), jnp.float32),
        pltpu.VMEM((H,), jnp.float32),
        pltpu.VMEM_SHARED((NP, H), jnp.float32),
    ],
)
def _sc_prop2(e10_hbm, e11_hbm, disw_hbm, b1_hbm, src_hbm, dst_hbm,
              zeros_hbm, out0_hbm, out1_hbm,
              src_v, dst_v, srce_v, dste_v, rows_v, acc_sh, gsem, ssem,
              t0_v, t1_v, dw_v, b1_v, hp2_sh):
    c = lax.axis_index("c")
    s = lax.axis_index("s")
    w = c * NS + s
    pltpu.sync_copy(e10_hbm.at[pl.ds(s * RPS, RPS)], t0_v)
    pltpu.sync_copy(e11_hbm.at[pl.ds(s * RPS, RPS)], t1_v)
    pltpu.sync_copy(disw_hbm.at[pl.ds(s * RPS, RPS)], dw_v)
    pltpu.sync_copy(b1_hbm, b1_v)
    pltpu.sync_copy(src_hbm.at[pl.ds(w * CPW, CPW)], src_v)
    pltpu.sync_copy(dst_hbm.at[pl.ds(w * CPW, CPW)], dst_v)

    @pl.when(w < NX)
    def _():
        pltpu.sync_copy(src_hbm.at[pl.ds(NW * CPW + w, 1)], srce_v)
        pltpu.sync_copy(dst_hbm.at[pl.ds(NW * CPW + w, 1)], dste_v)

    b1r = b1_v[...]

    def mid_body(r, carry):
        dw = dw_v[r]
        h = jnp.maximum((t0_v[r] + t1_v[r]) * dw + b1r, 0.0) * dw
        t0_v[r] = h
        return carry

    lax.fori_loop(0, RPS, mid_body, 0)
    pltpu.sync_copy(t0_v, hp2_sh.at[pl.ds(s * RPS, RPS)])

    @pl.when(c == 0)
    def _():
        pltpu.sync_copy(t0_v, acc_sh.at[pl.ds(s * RPS, RPS)])

    @pl.when(c == 1)
    def _():
        pltpu.sync_copy(zeros_hbm.at[pl.ds(s * RPS, RPS)],
                        acc_sh.at[pl.ds(s * RPS, RPS)])

    plsc.subcore_barrier()
    _edge_phase(hp2_sh, src_v, dst_v, srce_v, dste_v, rows_v, acc_sh,
                gsem, ssem, w)
    plsc.subcore_barrier()

    @pl.when(c == 0)
    def _():
        pltpu.sync_copy(acc_sh.at[pl.ds(s * RPS, RPS)],
                        out0_hbm.at[pl.ds(s * RPS, RPS)])

    @pl.when(c == 1)
    def _():
        pltpu.sync_copy(acc_sh.at[pl.ds(s * RPS, RPS)],
                        out1_hbm.at[pl.ds(s * RPS, RPS)])


R = 2000
G = N // R


def _tc_first_body(x_ref, w1_ref, d0_ref, d1_ref, hp_ref, disw_ref):
    dis = lax.rsqrt(d0_ref[...] + d1_ref[...] + 1.0)
    h = jnp.dot(x_ref[...], w1_ref[...], preferred_element_type=jnp.float32)
    hp_ref[...] = dis * h
    disw_ref[...] = dis


def _tc_final_body(e0_ref, e1_ref, disw_ref, w2_ref, b2_ref, out_ref):
    t = disw_ref[...] * (e0_ref[...] + e1_ref[...])
    out_ref[...] = (jnp.dot(t, w2_ref[...], preferred_element_type=jnp.float32)
                    + b2_ref[...])


def _row_spec(width):
    return pl.BlockSpec((R, width), lambda i: (i, 0))


def _full_spec(h, width):
    return pl.BlockSpec((h, width), lambda i: (0, 0))


def _tc_first(x, W1, d0, d1):
    return pl.pallas_call(
        _tc_first_body,
        grid=(G,),
        in_specs=[_row_spec(128), _full_spec(128, H), _row_spec(H),
                  _row_spec(H)],
        out_specs=(_row_spec(H), _row_spec(H)),
        out_shape=(jax.ShapeDtypeStruct((NP, H), jnp.float32),
                   jax.ShapeDtypeStruct((NP, H), jnp.float32)),
    )(x, W1, d0, d1)


def _tc_final(e0, e1, disw, W2, b2):
    return pl.pallas_call(
        _tc_final_body,
        grid=(G,),
        in_specs=[_row_spec(H)] * 3 + [_full_spec(H, 128), _full_spec(1, 128)],
        out_specs=_row_spec(128),
        out_shape=jax.ShapeDtypeStruct((N, 128), jnp.float32),
    )(e0, e1, disw, W2, b2)


def kernel(x, edge_index, W1, b1, W2, b2):
    ei = edge_index.astype(jnp.int32)
    src = ei[0].reshape(NCHT, CH)
    dst = ei[1].reshape(NCHT, CH)
    zeros_p = jnp.zeros((NP, H), jnp.float32)
    ones_c = jnp.ones((CH, H), jnp.float32)

    d0, d1 = _sc_degree(dst, ones_c, zeros_p)
    hp1, disw = _tc_first(x, W1, d0, d1)
    e10, e11 = _sc_prop1(hp1, src, dst, zeros_p)
    e20, e21 = _sc_prop2(e10, e11, disw, b1.astype(jnp.float32), src, dst,
                         zeros_p)
    return _tc_final(e20, e21, disw, W2, b2.reshape(1, 128))


# speedup vs baseline: 69.9028x; 1.0752x over previous
import functools

import jax
import jax.numpy as jnp
from jax import lax
from jax.experimental import pallas as pl
from jax.experimental.pallas import tpu as pltpu
from jax.experimental.pallas import tpu_sc as plsc

N = 10000
H = 16
NC, NS = 2, 16
NW = NC * NS
NP = 10240
RPS = NP // NS
E = 320000
CH = 128
NCHT = E // CH
CPW = NCHT // NW
NX = NCHT - NW * CPW
GRP = 6
NG = CPW // GRP
GR = GRP * CH

_mesh = plsc.VectorSubcoreMesh(core_axis_name="c", subcore_axis_name="s")
_sc_params = pltpu.CompilerParams(use_tc_tiling_on_sc=False,
                                  needs_layout_passes=False)

_partial_out = (jax.ShapeDtypeStruct((NP, H), jnp.float32),
                jax.ShapeDtypeStruct((NP, H), jnp.float32))


@functools.partial(
    pl.kernel,
    out_type=_partial_out,
    mesh=_mesh,
    compiler_params=_sc_params,
    scratch_types=[
        pltpu.VMEM((CPW, CH), jnp.int32),
        pltpu.VMEM((1, CH), jnp.int32),
        pltpu.VMEM((CH, H), jnp.float32),
        pltpu.VMEM_SHARED((NP, H), jnp.float32),
        pltpu.SemaphoreType.DMA((2,)),
    ],
)
def _sc_degree(dst_hbm, ones_hbm, zeros_hbm, out0_hbm, out1_hbm,
               dst_v, dste_v, ones_v, acc_sh, ssem):
    c = lax.axis_index("c")
    s = lax.axis_index("s")
    w = c * NS + s
    pltpu.sync_copy(zeros_hbm.at[pl.ds(s * RPS, RPS)],
                    acc_sh.at[pl.ds(s * RPS, RPS)])
    pltpu.sync_copy(dst_hbm.at[pl.ds(w * CPW, CPW)], dst_v)
    pltpu.sync_copy(ones_hbm, ones_v)

    @pl.when(w < NX)
    def _():
        pltpu.sync_copy(dst_hbm.at[pl.ds(NW * CPW + w, 1)], dste_v)

    plsc.subcore_barrier()

    def body(j, carry):
        b = lax.rem(j, 2)
        for k in range(GRP):
            pltpu.async_copy(ones_v, acc_sh.at[dst_v.at[j * GRP + k]],
                             ssem.at[b], add=True)

        @pl.when(j >= 1)
        def _():
            for k in range(GRP):
                pltpu.make_async_copy(
                    ones_v, acc_sh.at[dst_v.at[(j - 1) * GRP + k]],
                    ssem.at[1 - b]).wait()

        return carry

    lax.fori_loop(0, NG, body, 0)
    for k in range(GRP):
        pltpu.make_async_copy(ones_v,
                              acc_sh.at[dst_v.at[(NG - 1) * GRP + k]],
                              ssem.at[(NG - 1) % 2]).wait()

    @pl.when(w < NX)
    def _():
        pltpu.sync_copy(ones_v, acc_sh.at[dste_v.at[0]], add=True)

    plsc.subcore_barrier()

    @pl.when(c == 0)
    def _():
        pltpu.sync_copy(acc_sh.at[pl.ds(s * RPS, RPS)],
                        out0_hbm.at[pl.ds(s * RPS, RPS)])

    @pl.when(c == 1)
    def _():
        pltpu.sync_copy(acc_sh.at[pl.ds(s * RPS, RPS)],
                        out1_hbm.at[pl.ds(s * RPS, RPS)])


def _edge_phase(table_sh, src_v, dst_v, srce_v, dste_v, rows_v, acc_sh,
                gsem, ssem, w):
    def _issue_gathers(g, b):
        for k in range(GRP):
            pltpu.async_copy(table_sh.at[src_v.at[g * GRP + k]],
                             rows_v.at[b].at[pl.ds(k * CH, CH)],
                             gsem.at[b])

    def _wait_gathers(g, b):
        for k in range(GRP):
            pltpu.make_async_copy(table_sh.at[src_v.at[g * GRP + k]],
                                  rows_v.at[b].at[pl.ds(k * CH, CH)],
                                  gsem.at[b]).wait()

    def _issue_scatters(g, b):
        for k in range(GRP):
            pltpu.async_copy(rows_v.at[b].at[pl.ds(k * CH, CH)],
                             acc_sh.at[dst_v.at[g * GRP + k]],
                             ssem.at[b], add=True)

    def _drain_scatters(g, b):
        for k in range(GRP):
            pltpu.make_async_copy(rows_v.at[b].at[pl.ds(k * CH, CH)],
                                  acc_sh.at[dst_v.at[g * GRP + k]],
                                  ssem.at[b]).wait()

    _issue_gathers(0, 0)

    def body(g, carry):
        b = lax.rem(g, 2)
        _wait_gathers(g, b)

        @pl.when(g >= 1)
        def _():
            _drain_scatters(g - 1, 1 - b)

        @pl.when(g + 1 < NG)
        def _():
            _issue_gathers(g + 1, 1 - b)

        _issue_scatters(g, b)
        return carry

    lax.fori_loop(0, NG, body, 0)
    _drain_scatters(NG - 1, (NG - 1) % 2)

    @pl.when(w < NX)
    def _():
        pltpu.sync_copy(table_sh.at[srce_v.at[0]],
                        rows_v.at[0].at[pl.ds(0, CH)])
        pltpu.sync_copy(rows_v.at[0].at[pl.ds(0, CH)],
                        acc_sh.at[dste_v.at[0]], add=True)


_prop_scratch = [
    pltpu.VMEM((CPW, CH), jnp.int32),
    pltpu.VMEM((CPW, CH), jnp.int32),
    pltpu.VMEM((1, CH), jnp.int32),
    pltpu.VMEM((1, CH), jnp.int32),
    pltpu.VMEM((2, GR, H), jnp.float32),
    pltpu.VMEM_SHARED((NP, H), jnp.float32),
    pltpu.VMEM_SHARED((NP, H), jnp.float32),
    pltpu.SemaphoreType.DMA((2,)),
    pltpu.SemaphoreType.DMA((2,)),
]

def _rsqrt_newton(d):
    y = plsc.bitcast(0x5F3759DF - lax.shift_right_arithmetic(
        plsc.bitcast(d, jnp.int32), 1), jnp.float32)
    for _ in range(3):
        y = y * (1.5 - 0.5 * d * y * y)
    return y


@functools.partial(
    pl.kernel,
    out_type=(jax.ShapeDtypeStruct((NP, H), jnp.float32),) * 3,
    mesh=_mesh,
    compiler_params=_sc_params,
    scratch_types=_prop_scratch + [
        pltpu.VMEM((RPS, H), jnp.float32),
        pltpu.VMEM((RPS, H), jnp.float32),
        pltpu.VMEM((RPS, H), jnp.float32),
    ],
)
def _sc_prop1(h1_hbm, d0_hbm, d1_hbm, src_hbm, dst_hbm, zeros_hbm,
              out0_hbm, out1_hbm, disw_hbm,
              src_v, dst_v, srce_v, dste_v, rows_v, acc_sh, hp_sh,
              gsem, ssem, t0_v, d0_v, d1_v):
    c = lax.axis_index("c")
    s = lax.axis_index("s")
    w = c * NS + s
    pltpu.sync_copy(h1_hbm.at[pl.ds(s * RPS, RPS)], t0_v)
    pltpu.sync_copy(d0_hbm.at[pl.ds(s * RPS, RPS)], d0_v)
    pltpu.sync_copy(d1_hbm.at[pl.ds(s * RPS, RPS)], d1_v)
    pltpu.sync_copy(src_hbm.at[pl.ds(w * CPW, CPW)], src_v)
    pltpu.sync_copy(dst_hbm.at[pl.ds(w * CPW, CPW)], dst_v)

    @pl.when(w < NX)
    def _():
        pltpu.sync_copy(src_hbm.at[pl.ds(NW * CPW + w, 1)], srce_v)
        pltpu.sync_copy(dst_hbm.at[pl.ds(NW * CPW + w, 1)], dste_v)

    def norm_body(r, carry):
        dis = _rsqrt_newton(d0_v[r] + d1_v[r] + 1.0)
        t0_v[r] = t0_v[r] * dis
        d0_v[r] = dis
        return carry

    lax.fori_loop(0, RPS, norm_body, 0)
    pltpu.sync_copy(t0_v, hp_sh.at[pl.ds(s * RPS, RPS)])

    @pl.when(c == 0)
    def _():
        pltpu.sync_copy(t0_v, acc_sh.at[pl.ds(s * RPS, RPS)])
        pltpu.sync_copy(d0_v, disw_hbm.at[pl.ds(s * RPS, RPS)])

    @pl.when(c == 1)
    def _():
        pltpu.sync_copy(zeros_hbm.at[pl.ds(s * RPS, RPS)],
                        acc_sh.at[pl.ds(s * RPS, RPS)])

    plsc.subcore_barrier()
    _edge_phase(hp_sh, src_v, dst_v, srce_v, dste_v, rows_v, acc_sh,
                gsem, ssem, w)
    plsc.subcore_barrier()

    @pl.when(c == 0)
    def _():
        pltpu.sync_copy(acc_sh.at[pl.ds(s * RPS, RPS)],
                        out0_hbm.at[pl.ds(s * RPS, RPS)])

    @pl.when(c == 1)
    def _():
        pltpu.sync_copy(acc_sh.at[pl.ds(s * RPS, RPS)],
                        out1_hbm.at[pl.ds(s * RPS, RPS)])


@functools.partial(
    pl.kernel,
    out_type=_partial_out,
    mesh=_mesh,
    compiler_params=_sc_params,
    scratch_types=_prop_scratch + [
        pltpu.VMEM((RPS, H), jnp.float32),
        pltpu.VMEM((RPS, H), jnp.float32),
        pltpu.VMEM((RPS, H), jnp.float32),
        pltpu.VMEM((H,), jnp.float32),
    ],
)
def _sc_prop2(e10_hbm, e11_hbm, disw_hbm, b1_hbm, src_hbm, dst_hbm,
              zeros_hbm, out0_hbm, out1_hbm,
              src_v, dst_v, srce_v, dste_v, rows_v, acc_sh, hp2_sh,
              gsem, ssem, t0_v, t1_v, dw_v, b1_v):
    c = lax.axis_index("c")
    s = lax.axis_index("s")
    w = c * NS + s
    pltpu.sync_copy(e10_hbm.at[pl.ds(s * RPS, RPS)], t0_v)
    pltpu.sync_copy(e11_hbm.at[pl.ds(s * RPS, RPS)], t1_v)
    pltpu.sync_copy(disw_hbm.at[pl.ds(s * RPS, RPS)], dw_v)
    pltpu.sync_copy(b1_hbm, b1_v)
    pltpu.sync_copy(src_hbm.at[pl.ds(w * CPW, CPW)], src_v)
    pltpu.sync_copy(dst_hbm.at[pl.ds(w * CPW, CPW)], dst_v)

    @pl.when(w < NX)
    def _():
        pltpu.sync_copy(src_hbm.at[pl.ds(NW * CPW + w, 1)], srce_v)
        pltpu.sync_copy(dst_hbm.at[pl.ds(NW * CPW + w, 1)], dste_v)

    b1r = b1_v[...]

    def mid_body(r, carry):
        dw = dw_v[r]
        h = jnp.maximum((t0_v[r] + t1_v[r]) * dw + b1r, 0.0) * dw
        t0_v[r] = h
        return carry

    lax.fori_loop(0, RPS, mid_body, 0)
    pltpu.sync_copy(t0_v, hp2_sh.at[pl.ds(s * RPS, RPS)])

    @pl.when(c == 0)
    def _():
        pltpu.sync_copy(t0_v, acc_sh.at[pl.ds(s * RPS, RPS)])

    @pl.when(c == 1)
    def _():
        pltpu.sync_copy(zeros_hbm.at[pl.ds(s * RPS, RPS)],
                        acc_sh.at[pl.ds(s * RPS, RPS)])

    plsc.subcore_barrier()
    _edge_phase(hp2_sh, src_v, dst_v, srce_v, dste_v, rows_v, acc_sh,
                gsem, ssem, w)
    plsc.subcore_barrier()

    @pl.when(c == 0)
    def _():
        pltpu.sync_copy(acc_sh.at[pl.ds(s * RPS, RPS)],
                        out0_hbm.at[pl.ds(s * RPS, RPS)])

    @pl.when(c == 1)
    def _():
        pltpu.sync_copy(acc_sh.at[pl.ds(s * RPS, RPS)],
                        out1_hbm.at[pl.ds(s * RPS, RPS)])


R = 2000
G = N // R


def _tc_matmul_body(x_ref, w1_ref, h_ref):
    h_ref[...] = jnp.dot(x_ref[...], w1_ref[...],
                         preferred_element_type=jnp.float32)


def _tc_final_body(e0_ref, e1_ref, disw_ref, w2_ref, b2_ref, out_ref):
    t = disw_ref[...] * (e0_ref[...] + e1_ref[...])
    out_ref[...] = (jnp.dot(t, w2_ref[...], preferred_element_type=jnp.float32)
                    + b2_ref[...])


def _row_spec(width):
    return pl.BlockSpec((R, width), lambda i: (i, 0))


def _full_spec(h, width):
    return pl.BlockSpec((h, width), lambda i: (0, 0))


def _tc_matmul(x, W1):
    return pl.pallas_call(
        _tc_matmul_body,
        grid=(G,),
        in_specs=[_row_spec(128), _full_spec(128, H)],
        out_specs=_row_spec(H),
        out_shape=jax.ShapeDtypeStruct((NP, H), jnp.float32),
    )(x, W1)


def _tc_final(e0, e1, disw, W2, b2):
    return pl.pallas_call(
        _tc_final_body,
        grid=(G,),
        in_specs=[_row_spec(H)] * 3 + [_full_spec(H, 128), _full_spec(1, 128)],
        out_specs=_row_spec(128),
        out_shape=jax.ShapeDtypeStruct((N, 128), jnp.float32),
    )(e0, e1, disw, W2, b2)


def kernel(x, edge_index, W1, b1, W2, b2):
    ei = edge_index.astype(jnp.int32)
    src = ei[0].reshape(NCHT, CH)
    dst = ei[1].reshape(NCHT, CH)
    zeros_p = jnp.zeros((NP, H), jnp.float32)
    ones_c = jnp.ones((CH, H), jnp.float32)

    h1 = _tc_matmul(x, W1)
    d0, d1 = _sc_degree(dst, ones_c, zeros_p)
    e10, e11, disw = _sc_prop1(h1, d0, d1, src, dst, zeros_p)
    e20, e21 = _sc_prop2(e10, e11, disw, b1.astype(jnp.float32), src, dst,
                         zeros_p)
    return _tc_final(e20, e21, disw, W2, b2.reshape(1, 128))


# speedup vs baseline: 78.8735x; 1.1283x over previous
import functools

import jax
import jax.numpy as jnp
from jax import lax
from jax.experimental import pallas as pl
from jax.experimental.pallas import tpu as pltpu
from jax.experimental.pallas import tpu_sc as plsc

N = 10000
H = 16
NC, NS = 2, 16
NW = NC * NS
NP = 10240
RPS = NP // NS
E = 320000
CH = 128
NCHT = E // CH
CPW = NCHT // NW
NX = NCHT - NW * CPW
GRP = 6
NG = CPW // GRP
GR = GRP * CH

_mesh = plsc.VectorSubcoreMesh(core_axis_name="c", subcore_axis_name="s")
_sc_params = pltpu.CompilerParams(use_tc_tiling_on_sc=False,
                                  needs_layout_passes=False)

_partial_out = (jax.ShapeDtypeStruct((NP, H), jnp.float32),
                jax.ShapeDtypeStruct((NP, H), jnp.float32))


@functools.partial(
    pl.kernel,
    out_type=_partial_out,
    mesh=_mesh,
    compiler_params=_sc_params,
    scratch_types=[
        pltpu.VMEM((CPW, CH), jnp.int32),
        pltpu.VMEM((1, CH), jnp.int32),
        pltpu.VMEM((CH, H), jnp.float32),
        pltpu.VMEM_SHARED((NP, H), jnp.float32),
        pltpu.SemaphoreType.DMA((2,)),
    ],
)
def _sc_degree(dst_hbm, ones_hbm, zeros_hbm, out0_hbm, out1_hbm,
               dst_v, dste_v, ones_v, acc_sh, ssem):
    c = lax.axis_index("c")
    s = lax.axis_index("s")
    w = c * NS + s
    pltpu.sync_copy(zeros_hbm.at[pl.ds(s * RPS, RPS)],
                    acc_sh.at[pl.ds(s * RPS, RPS)])
    pltpu.sync_copy(dst_hbm.at[pl.ds(w * CPW, CPW)], dst_v)
    pltpu.sync_copy(ones_hbm, ones_v)

    @pl.when(w < NX)
    def _():
        pltpu.sync_copy(dst_hbm.at[pl.ds(NW * CPW + w, 1)], dste_v)

    plsc.subcore_barrier()

    def body(j, carry):
        b = lax.rem(j, 2)
        for k in range(GRP):
            pltpu.async_copy(ones_v, acc_sh.at[dst_v.at[j * GRP + k]],
                             ssem.at[b], add=True)

        @pl.when(j >= 1)
        def _():
            for k in range(GRP):
                pltpu.make_async_copy(
                    ones_v, acc_sh.at[dst_v.at[(j - 1) * GRP + k]],
                    ssem.at[1 - b]).wait()

        return carry

    lax.fori_loop(0, NG, body, 0)
    for k in range(GRP):
        pltpu.make_async_copy(ones_v,
                              acc_sh.at[dst_v.at[(NG - 1) * GRP + k]],
                              ssem.at[(NG - 1) % 2]).wait()

    @pl.when(w < NX)
    def _():
        pltpu.sync_copy(ones_v, acc_sh.at[dste_v.at[0]], add=True)

    plsc.subcore_barrier()

    @pl.when(c == 0)
    def _():
        pltpu.sync_copy(acc_sh.at[pl.ds(s * RPS, RPS)],
                        out0_hbm.at[pl.ds(s * RPS, RPS)])

    @pl.when(c == 1)
    def _():
        pltpu.sync_copy(acc_sh.at[pl.ds(s * RPS, RPS)],
                        out1_hbm.at[pl.ds(s * RPS, RPS)])


def _edge_phase(table_sh, src_v, dst_v, srce_v, dste_v, rows_v, acc_sh,
                gsem, ssem, w):
    def _issue_gathers(g, b):
        for k in range(GRP):
            pltpu.async_copy(table_sh.at[src_v.at[g * GRP + k]],
                             rows_v.at[b].at[pl.ds(k * CH, CH)],
                             gsem.at[b])

    def _wait_gathers(g, b):
        for k in range(GRP):
            pltpu.make_async_copy(table_sh.at[src_v.at[g * GRP + k]],
                                  rows_v.at[b].at[pl.ds(k * CH, CH)],
                                  gsem.at[b]).wait()

    def _issue_scatters(g, b):
        for k in range(GRP):
            pltpu.async_copy(rows_v.at[b].at[pl.ds(k * CH, CH)],
                             acc_sh.at[dst_v.at[g * GRP + k]],
                             ssem.at[b], add=True)

    def _drain_scatters(g, b):
        for k in range(GRP):
            pltpu.make_async_copy(rows_v.at[b].at[pl.ds(k * CH, CH)],
                                  acc_sh.at[dst_v.at[g * GRP + k]],
                                  ssem.at[b]).wait()

    _issue_gathers(0, 0)

    def body(g, carry):
        b = lax.rem(g, 2)
        _wait_gathers(g, b)

        @pl.when(g >= 1)
        def _():
            _drain_scatters(g - 1, 1 - b)

        @pl.when(g + 1 < NG)
        def _():
            _issue_gathers(g + 1, 1 - b)

        _issue_scatters(g, b)
        return carry

    lax.fori_loop(0, NG, body, 0)
    _drain_scatters(NG - 1, (NG - 1) % 2)

    @pl.when(w < NX)
    def _():
        pltpu.sync_copy(table_sh.at[srce_v.at[0]],
                        rows_v.at[0].at[pl.ds(0, CH)])
        pltpu.sync_copy(rows_v.at[0].at[pl.ds(0, CH)],
                        acc_sh.at[dste_v.at[0]], add=True)


_prop_scratch = [
    pltpu.VMEM((CPW, CH), jnp.int32),
    pltpu.VMEM((CPW, CH), jnp.int32),
    pltpu.VMEM((1, CH), jnp.int32),
    pltpu.VMEM((1, CH), jnp.int32),
    pltpu.VMEM((2, GR, H), jnp.float32),
    pltpu.VMEM_SHARED((NP, H), jnp.float32),
    pltpu.VMEM_SHARED((NP, H), jnp.float32),
    pltpu.SemaphoreType.DMA((2,)),
    pltpu.SemaphoreType.DMA((2,)),
]

def _rsqrt_newton(d):
    y = plsc.bitcast(0x5F3759DF - lax.shift_right_arithmetic(
        plsc.bitcast(d, jnp.int32), 1), jnp.float32)
    for _ in range(2):
        y = y * (1.5 - 0.5 * d * y * y)
    return y


@functools.partial(
    pl.kernel,
    out_type=(jax.ShapeDtypeStruct((NP, H), jnp.float32),) * 3,
    mesh=_mesh,
    compiler_params=_sc_params,
    scratch_types=_prop_scratch + [
        pltpu.VMEM((RPS, H), jnp.float32),
        pltpu.VMEM((RPS, H), jnp.float32),
        pltpu.VMEM((RPS, H), jnp.float32),
    ],
)
def _sc_prop1(h1_hbm, d0_hbm, d1_hbm, src_hbm, dst_hbm, zeros_hbm,
              out0_hbm, out1_hbm, disw_hbm,
              src_v, dst_v, srce_v, dste_v, rows_v, acc_sh, hp_sh,
              gsem, ssem, t0_v, d0_v, d1_v):
    c = lax.axis_index("c")
    s = lax.axis_index("s")
    w = c * NS + s
    pltpu.sync_copy(h1_hbm.at[pl.ds(s * RPS, RPS)], t0_v)
    pltpu.sync_copy(d0_hbm.at[pl.ds(s * RPS, RPS)], d0_v)
    pltpu.sync_copy(d1_hbm.at[pl.ds(s * RPS, RPS)], d1_v)
    pltpu.sync_copy(src_hbm.at[pl.ds(w * CPW, CPW)], src_v)
    pltpu.sync_copy(dst_hbm.at[pl.ds(w * CPW, CPW)], dst_v)

    @pl.when(w < NX)
    def _():
        pltpu.sync_copy(src_hbm.at[pl.ds(NW * CPW + w, 1)], srce_v)
        pltpu.sync_copy(dst_hbm.at[pl.ds(NW * CPW + w, 1)], dste_v)

    @plsc.parallel_loop(0, RPS, step=1, unroll=8)
    def _(r):
        dis = _rsqrt_newton(d0_v[r] + d1_v[r] + 1.0)
        t0_v[r] = t0_v[r] * dis
        d0_v[r] = dis
    pltpu.sync_copy(t0_v, hp_sh.at[pl.ds(s * RPS, RPS)])

    @pl.when(c == 0)
    def _():
        pltpu.sync_copy(t0_v, acc_sh.at[pl.ds(s * RPS, RPS)])
        pltpu.sync_copy(d0_v, disw_hbm.at[pl.ds(s * RPS, RPS)])

    @pl.when(c == 1)
    def _():
        pltpu.sync_copy(zeros_hbm.at[pl.ds(s * RPS, RPS)],
                        acc_sh.at[pl.ds(s * RPS, RPS)])

    plsc.subcore_barrier()
    _edge_phase(hp_sh, src_v, dst_v, srce_v, dste_v, rows_v, acc_sh,
                gsem, ssem, w)
    plsc.subcore_barrier()

    @pl.when(c == 0)
    def _():
        pltpu.sync_copy(acc_sh.at[pl.ds(s * RPS, RPS)],
                        out0_hbm.at[pl.ds(s * RPS, RPS)])

    @pl.when(c == 1)
    def _():
        pltpu.sync_copy(acc_sh.at[pl.ds(s * RPS, RPS)],
                        out1_hbm.at[pl.ds(s * RPS, RPS)])


@functools.partial(
    pl.kernel,
    out_type=_partial_out,
    mesh=_mesh,
    compiler_params=_sc_params,
    scratch_types=_prop_scratch + [
        pltpu.VMEM((RPS, H), jnp.float32),
        pltpu.VMEM((RPS, H), jnp.float32),
        pltpu.VMEM((RPS, H), jnp.float32),
        pltpu.VMEM((H,), jnp.float32),
    ],
)
def _sc_prop2(e10_hbm, e11_hbm, disw_hbm, b1_hbm, src_hbm, dst_hbm,
              zeros_hbm, out0_hbm, out1_hbm,
              src_v, dst_v, srce_v, dste_v, rows_v, acc_sh, hp2_sh,
              gsem, ssem, t0_v, t1_v, dw_v, b1_v):
    c = lax.axis_index("c")
    s = lax.axis_index("s")
    w = c * NS + s
    pltpu.sync_copy(e10_hbm.at[pl.ds(s * RPS, RPS)], t0_v)
    pltpu.sync_copy(e11_hbm.at[pl.ds(s * RPS, RPS)], t1_v)
    pltpu.sync_copy(disw_hbm.at[pl.ds(s * RPS, RPS)], dw_v)
    pltpu.sync_copy(b1_hbm, b1_v)
    pltpu.sync_copy(src_hbm.at[pl.ds(w * CPW, CPW)], src_v)
    pltpu.sync_copy(dst_hbm.at[pl.ds(w * CPW, CPW)], dst_v)

    @pl.when(w < NX)
    def _():
        pltpu.sync_copy(src_hbm.at[pl.ds(NW * CPW + w, 1)], srce_v)
        pltpu.sync_copy(dst_hbm.at[pl.ds(NW * CPW + w, 1)], dste_v)

    b1r = b1_v[...]

    @plsc.parallel_loop(0, RPS, step=1, unroll=8)
    def _(r):
        dw = dw_v[r]
        t0_v[r] = jnp.maximum((t0_v[r] + t1_v[r]) * dw + b1r, 0.0) * dw
    pltpu.sync_copy(t0_v, hp2_sh.at[pl.ds(s * RPS, RPS)])

    @pl.when(c == 0)
    def _():
        pltpu.sync_copy(t0_v, acc_sh.at[pl.ds(s * RPS, RPS)])

    @pl.when(c == 1)
    def _():
        pltpu.sync_copy(zeros_hbm.at[pl.ds(s * RPS, RPS)],
                        acc_sh.at[pl.ds(s * RPS, RPS)])

    plsc.subcore_barrier()
    _edge_phase(hp2_sh, src_v, dst_v, srce_v, dste_v, rows_v, acc_sh,
                gsem, ssem, w)
    plsc.subcore_barrier()

    @pl.when(c == 0)
    def _():
        pltpu.sync_copy(acc_sh.at[pl.ds(s * RPS, RPS)],
                        out0_hbm.at[pl.ds(s * RPS, RPS)])

    @pl.when(c == 1)
    def _():
        pltpu.sync_copy(acc_sh.at[pl.ds(s * RPS, RPS)],
                        out1_hbm.at[pl.ds(s * RPS, RPS)])


R = 2000
G = N // R


def _tc_matmul_body(x_ref, w1_ref, h_ref):
    h_ref[...] = jnp.dot(x_ref[...], w1_ref[...],
                         preferred_element_type=jnp.float32)


def _tc_final_body(e0_ref, e1_ref, disw_ref, w2_ref, b2_ref, out_ref):
    t = disw_ref[...] * (e0_ref[...] + e1_ref[...])
    out_ref[...] = (jnp.dot(t, w2_ref[...], preferred_element_type=jnp.float32)
                    + b2_ref[...])


def _row_spec(width):
    return pl.BlockSpec((R, width), lambda i: (i, 0))


def _full_spec(h, width):
    return pl.BlockSpec((h, width), lambda i: (0, 0))


def _tc_matmul(x, W1):
    return pl.pallas_call(
        _tc_matmul_body,
        grid=(G,),
        in_specs=[_row_spec(128), _full_spec(128, H)],
        out_specs=_row_spec(H),
        out_shape=jax.ShapeDtypeStruct((NP, H), jnp.float32),
    )(x, W1)


def _tc_final(e0, e1, disw, W2, b2):
    return pl.pallas_call(
        _tc_final_body,
        grid=(G,),
        in_specs=[_row_spec(H)] * 3 + [_full_spec(H, 128), _full_spec(1, 128)],
        out_specs=_row_spec(128),
        out_shape=jax.ShapeDtypeStruct((N, 128), jnp.float32),
    )(e0, e1, disw, W2, b2)


def kernel(x, edge_index, W1, b1, W2, b2):
    ei = edge_index.astype(jnp.int32)
    src = ei[0].reshape(NCHT, CH)
    dst = ei[1].reshape(NCHT, CH)
    zeros_p = jnp.zeros((NP, H), jnp.float32)
    ones_c = jnp.ones((CH, H), jnp.float32)

    h1 = _tc_matmul(x, W1)
    d0, d1 = _sc_degree(dst, ones_c, zeros_p)
    e10, e11, disw = _sc_prop1(h1, d0, d1, src, dst, zeros_p)
    e20, e21 = _sc_prop2(e10, e11, disw, b1.astype(jnp.float32), src, dst,
                         zeros_p)
    return _tc_final(e20, e21, disw, W2, b2.reshape(1, 128))
